# Initial kernel scaffold; baseline (speedup 1.0000x reference)
import jax
import jax.numpy as jnp
from jax.experimental import pallas as pl


def kernel(x, edge_index, edge_weight, W1, b1, W2, b2, W3, b3, M1w, M1b, g1, be1, M2w, M2b, g2, be2, M3w, M3b, g3, be3, M4w, M4b):
    raise NotImplementedError("write your pallas kernel here")


# speedup vs baseline: 1.0033x; 1.0033x over previous
import jax
import jax.numpy as jnp
from jax.experimental import pallas as pl

N = 10000
E = 320000
H = 256
BE = 6400
NB = E // BE


def _p1_body(sa_ref, sb_ref, ew_ref, c_ref, b_ref, h_ref, s_ref, q_ref):
    i = pl.program_id(0)
    h = sa_ref[...] + sb_ref[...] + ew_ref[...] * c_ref[...] + b_ref[...]
    h_ref[...] = h

    @pl.when(i == 0)
    def _():
        s_ref[...] = jnp.zeros_like(s_ref)
        q_ref[...] = jnp.zeros_like(q_ref)

    s_ref[...] += jnp.sum(h, axis=0, keepdims=True)
    q_ref[...] += jnp.sum(h * h, axis=0, keepdims=True)


def _p23_body(h_ref, s_in, q_in, g_ref, be_ref, w_ref, b_ref,
              out_ref, s_ref, q_ref):
    i = pl.program_id(0)
    mu = s_in[...] / E
    var = q_in[...] / E - mu * mu
    a = g_ref[...] * jax.lax.rsqrt(var + 1e-5)
    sh = be_ref[...] - mu * a
    e = jnp.maximum(h_ref[...] * a + sh, 0.0)
    h2 = jnp.dot(e, w_ref[...], preferred_element_type=jnp.float32) + b_ref[...]
    out_ref[...] = h2

    @pl.when(i == 0)
    def _():
        s_ref[...] = jnp.zeros_like(s_ref)
        q_ref[...] = jnp.zeros_like(q_ref)

    s_ref[...] += jnp.sum(h2, axis=0, keepdims=True)
    q_ref[...] += jnp.sum(h2 * h2, axis=0, keepdims=True)


def _p4_body(h_ref, s_in, q_in, g_ref, be_ref, w_ref, b0_ref, out_ref):
    mu = s_in[...] / E
    var = q_in[...] / E - mu * mu
    a = g_ref[...] * jax.lax.rsqrt(var + 1e-5)
    sh = be_ref[...] - mu * a
    e = jnp.maximum(h_ref[...] * a + sh, 0.0)
    out_ref[...] = jnp.sum(e * w_ref[...], axis=1, keepdims=True) + b0_ref[0, 0]


def _row_spec():
    return pl.BlockSpec((1, H), lambda i: (0, 0))


def _blk_spec():
    return pl.BlockSpec((BE, H), lambda i: (i, 0))


def _edge_mlp(SA, SB, ew, M1c, M1b, g1, be1, M2w, M2b, g2, be2,
              M3w, M3b, g3, be3, M4w, M4b):
    f32 = jnp.float32
    h1, s1, q1 = pl.pallas_call(
        _p1_body,
        grid=(NB,),
        in_specs=[_blk_spec(), _blk_spec(),
                  pl.BlockSpec((BE, 1), lambda i: (i, 0)),
                  _row_spec(), _row_spec()],
        out_specs=[_blk_spec(), _row_spec(), _row_spec()],
        out_shape=[jax.ShapeDtypeStruct((E, H), f32),
                   jax.ShapeDtypeStruct((1, H), f32),
                   jax.ShapeDtypeStruct((1, H), f32)],
    )(SA, SB, ew.reshape(E, 1), M1c.reshape(1, H), M1b.reshape(1, H))

    def mid(h, s, q, g, be, w, b):
        return pl.pallas_call(
            _p23_body,
            grid=(NB,),
            in_specs=[_blk_spec(), _row_spec(), _row_spec(), _row_spec(),
                      _row_spec(), pl.BlockSpec((H, H), lambda i: (0, 0)),
                      _row_spec()],
            out_specs=[_blk_spec(), _row_spec(), _row_spec()],
            out_shape=[jax.ShapeDtypeStruct((E, H), f32),
                       jax.ShapeDtypeStruct((1, H), f32),
                       jax.ShapeDtypeStruct((1, H), f32)],
        )(h, s, q, g.reshape(1, H), be.reshape(1, H), w, b.reshape(1, H))

    h2, s2, q2 = mid(h1, s1, q1, g1, be1, M2w, M2b)
    h3, s3, q3 = mid(h2, s2, q2, g2, be2, M3w, M3b)

    logits = pl.pallas_call(
        _p4_body,
        grid=(NB,),
        in_specs=[_blk_spec(), _row_spec(), _row_spec(), _row_spec(),
                  _row_spec(), _row_spec(),
                  pl.BlockSpec((1, 1), lambda i: (0, 0))],
        out_specs=pl.BlockSpec((BE, 1), lambda i: (i, 0)),
        out_shape=jax.ShapeDtypeStruct((E, 1), f32),
    )(h3, s3, q3, g3.reshape(1, H), be3.reshape(1, H),
      M4w.reshape(1, H), M4b.reshape(1, 1))
    return logits.reshape(E)


def kernel(x, edge_index, edge_weight, W1, b1, W2, b2, W3, b3,
           M1w, M1b, g1, be1, M2w, M2b, g2, be2, M3w, M3b, g3, be3,
           M4w, M4b):
    src, dst = edge_index[0], edge_index[1]
    ew = edge_weight

    def gcn(h, W, b, dinv):
        hW = h @ W
        norm = dinv[src] * ew * dinv[dst]
        out = jnp.zeros_like(hW).at[dst].add(norm[:, None] * hW[src])
        return out + (dinv * dinv)[:, None] * hW + b

    deg = jnp.zeros((N,), jnp.float32).at[dst].add(ew) + 1.0
    dinv = jax.lax.rsqrt(deg)
    h = jax.nn.relu(gcn(x, W1, b1, dinv))
    h = jax.nn.relu(gcn(h, W2, b2, dinv))
    node = gcn(h, W3, b3, dinv)

    nodeA = node @ M1w[:H]
    nodeB = node @ M1w[H:2 * H]
    M1c = M1w[2 * H]
    SA = nodeA[src]
    SB = nodeB[dst]
    return _edge_mlp(SA, SB, ew, M1c, M1b, g1, be1, M2w, M2b, g2, be2,
                     M3w, M3b, g3, be3, M4w, M4b)


# speedup vs baseline: 1.6018x; 1.5965x over previous
import dataclasses
import functools

import jax
import jax.numpy as jnp
from jax import lax
from jax.experimental import pallas as pl
from jax.experimental.pallas import tpu as pltpu
from jax.experimental.pallas import tpu_sc as plsc

N = 10000
NPAD = 10240
E = 320000
H = 256
NTILES = 32
FP = 4
NPASS = H // (NTILES * FP)
CH = 2000
EDGT = E // NTILES
BE = 6400
NB = E // BE

_mesh = plsc.VectorSubcoreMesh(core_axis_name="c", subcore_axis_name="s")


def _cp():
    cp = pltpu.CompilerParams()
    if "needs_layout_passes" in pltpu.CompilerParams.__dataclass_fields__:
        cp = dataclasses.replace(cp, needs_layout_passes=False)
    return cp


def _deg_partials(dst, ew):
    @functools.partial(
        pl.kernel, mesh=_mesh, compiler_params=_cp(),
        out_type=jax.ShapeDtypeStruct((NTILES, NPAD), jnp.float32),
        scratch_types=[pltpu.VMEM((NPAD,), jnp.float32),
                       pltpu.VMEM((CH,), jnp.int32),
                       pltpu.VMEM((CH,), jnp.float32)])
    def k(dst_hbm, ew_hbm, out_hbm, accv, dv, wv):
        wid = lax.axis_index("s") * 2 + lax.axis_index("c")

        @pl.loop(0, NPAD, step=16)
        def _(j):
            accv[pl.ds(j, 16)] = jnp.zeros((16,), jnp.float32)

        base = wid * EDGT

        @pl.loop(0, EDGT, step=CH)
        def _(c):
            pltpu.sync_copy(dst_hbm.at[pl.ds(base + c, CH)], dv)
            pltpu.sync_copy(ew_hbm.at[pl.ds(base + c, CH)], wv)

            @pl.loop(0, CH, step=16)
            def _(g):
                plsc.addupdate_scatter(accv, [dv[pl.ds(g, 16)]],
                                       wv[pl.ds(g, 16)])

        pltpu.sync_copy(accv, out_hbm.at[wid])

    return k(dst, ew)


def _sc_aggregate(gT, src, dst, ew):
    @functools.partial(
        pl.kernel, mesh=_mesh, compiler_params=_cp(),
        out_type=jax.ShapeDtypeStruct((H, NPAD), jnp.float32),
        scratch_types=[pltpu.VMEM((FP, NPAD), jnp.float32),
                       pltpu.VMEM((FP, NPAD), jnp.float32),
                       pltpu.VMEM((CH,), jnp.int32),
                       pltpu.VMEM((CH,), jnp.int32),
                       pltpu.VMEM((CH,), jnp.float32)])
    def k(gT_hbm, src_hbm, dst_hbm, ew_hbm, out_hbm, gv, accv, sv, dv, wv):
        wid = lax.axis_index("s") * 2 + lax.axis_index("c")
        for p in range(NPASS):
            fbase = (p * NTILES + wid) * FP
            pltpu.sync_copy(gT_hbm.at[pl.ds(fbase, FP)], gv)
            for f in range(FP):
                @pl.loop(0, NPAD, step=16)
                def _(j, f=f):
                    accv[f, pl.ds(j, 16)] = jnp.zeros((16,), jnp.float32)

            @pl.loop(0, E, step=CH)
            def _(c):
                pltpu.sync_copy(src_hbm.at[pl.ds(c, CH)], sv)
                pltpu.sync_copy(dst_hbm.at[pl.ds(c, CH)], dv)
                pltpu.sync_copy(ew_hbm.at[pl.ds(c, CH)], wv)

                @pl.loop(0, CH, step=16)
                def _(g):
                    s16 = sv[pl.ds(g, 16)]
                    d16 = dv[pl.ds(g, 16)]
                    w16 = wv[pl.ds(g, 16)]
                    for f in range(FP):
                        f16 = jnp.full((16,), f, jnp.int32)
                        vals = plsc.load_gather(gv, [f16, s16]) * w16
                        plsc.addupdate_scatter(accv, [f16, d16], vals)

            pltpu.sync_copy(accv, out_hbm.at[pl.ds(fbase, FP)])

    return k(gT, src, dst, ew)


def _sc_gather_rows(tT, idx):
    FP8 = 8
    CHS = 1280

    @functools.partial(
        pl.kernel, mesh=_mesh, compiler_params=_cp(),
        out_type=jax.ShapeDtypeStruct((H, E), jnp.float32),
        scratch_types=[pltpu.VMEM((FP8, NPAD), jnp.float32),
                       pltpu.VMEM((CHS,), jnp.int32),
                       pltpu.VMEM((FP8, CHS), jnp.float32)])
    def k(t_hbm, i_hbm, out_hbm, tv, iv, ov):
        wid = lax.axis_index("s") * 2 + lax.axis_index("c")
        fbase = wid * FP8
        pltpu.sync_copy(t_hbm.at[pl.ds(fbase, FP8)], tv)

        @pl.loop(0, E, step=CHS)
        def _(c):
            pltpu.sync_copy(i_hbm.at[pl.ds(c, CHS)], iv)

            @pl.loop(0, CHS, step=16)
            def _(g):
                i16 = iv[pl.ds(g, 16)]
                for f in range(FP8):
                    f16 = jnp.full((16,), f, jnp.int32)
                    ov[f, pl.ds(g, 16)] = plsc.load_gather(tv, [f16, i16])

            pltpu.sync_copy(ov, out_hbm.at[pl.ds(fbase, FP8), pl.ds(c, CHS)])

    return k(tT, idx)


def _tk_matmul(wT, xT):
    def body(w_ref, x_ref, o_ref):
        o_ref[...] = jnp.dot(w_ref[...], x_ref[...],
                             preferred_element_type=jnp.float32)
    return pl.pallas_call(
        body, out_shape=jax.ShapeDtypeStruct((wT.shape[0], xT.shape[1]),
                                             jnp.float32))(wT, xT)


def _tk_dinv(degp, t1T):
    def body(dp_ref, t_ref, dinv_ref, g_ref):
        deg = jnp.sum(dp_ref[...], axis=0, keepdims=True) + 1.0
        dinv = jax.lax.rsqrt(deg)
        dinv_ref[...] = dinv
        g_ref[...] = t_ref[...] * dinv
    return pl.pallas_call(
        body, out_shape=[jax.ShapeDtypeStruct((1, NPAD), jnp.float32),
                         jax.ShapeDtypeStruct((H, NPAD), jnp.float32)])(degp, t1T)


def _tk_mid(agg, tT, dinv, b, wT):
    def body(a_ref, t_ref, dinv_ref, b_ref, w_ref, tn_ref, gn_ref):
        dinv = dinv_ref[...]
        h = jnp.maximum(dinv * a_ref[...] + dinv * dinv * t_ref[...]
                        + b_ref[...], 0.0)
        tn = jnp.dot(w_ref[...], h, preferred_element_type=jnp.float32)
        tn_ref[...] = tn
        gn_ref[...] = tn * dinv
    return pl.pallas_call(
        body, out_shape=[jax.ShapeDtypeStruct((H, NPAD), jnp.float32),
                         jax.ShapeDtypeStruct((H, NPAD), jnp.float32)])(
        agg, tT, dinv, b, wT)


def _tk_node_tables(agg, tT, dinv, b, waT, wbT):
    def body(a_ref, t_ref, dinv_ref, b_ref, wa_ref, wb_ref, oa_ref, ob_ref):
        dinv = dinv_ref[...]
        node = dinv * a_ref[...] + dinv * dinv * t_ref[...] + b_ref[...]
        oa_ref[...] = jnp.dot(wa_ref[...], node,
                              preferred_element_type=jnp.float32)
        ob_ref[...] = jnp.dot(wb_ref[...], node,
                              preferred_element_type=jnp.float32)
    return pl.pallas_call(
        body, out_shape=[jax.ShapeDtypeStruct((H, NPAD), jnp.float32),
                         jax.ShapeDtypeStruct((H, NPAD), jnp.float32)])(
        agg, tT, dinv, b, waT, wbT)


def _col_spec():
    return pl.BlockSpec((H, 1), lambda i: (0, 0))


def _eblk_spec():
    return pl.BlockSpec((H, BE), lambda i: (0, i))


def _p1_body(sa_ref, sb_ref, ew_ref, c_ref, b_ref, h_ref, s_ref, q_ref):
    i = pl.program_id(0)
    h = sa_ref[...] + sb_ref[...] + c_ref[...] * ew_ref[...] + b_ref[...]
    h_ref[...] = h

    @pl.when(i == 0)
    def _():
        s_ref[...] = jnp.zeros_like(s_ref)
        q_ref[...] = jnp.zeros_like(q_ref)

    s_ref[...] += jnp.sum(h, axis=1, keepdims=True)
    q_ref[...] += jnp.sum(h * h, axis=1, keepdims=True)


def _p23_body(h_ref, s_in, q_in, g_ref, be_ref, w_ref, b_ref,
              out_ref, s_ref, q_ref):
    i = pl.program_id(0)
    mu = s_in[...] / E
    var = q_in[...] / E - mu * mu
    a = g_ref[...] * jax.lax.rsqrt(var + 1e-5)
    sh = be_ref[...] - mu * a
    e = jnp.maximum(h_ref[...] * a + sh, 0.0)
    h2 = jnp.dot(w_ref[...], e, preferred_element_type=jnp.float32) + b_ref[...]
    out_ref[...] = h2

    @pl.when(i == 0)
    def _():
        s_ref[...] = jnp.zeros_like(s_ref)
        q_ref[...] = jnp.zeros_like(q_ref)

    s_ref[...] += jnp.sum(h2, axis=1, keepdims=True)
    q_ref[...] += jnp.sum(h2 * h2, axis=1, keepdims=True)


def _p4_body(h_ref, s_in, q_in, g_ref, be_ref, w_ref, b0_ref, out_ref):
    mu = s_in[...] / E
    var = q_in[...] / E - mu * mu
    a = g_ref[...] * jax.lax.rsqrt(var + 1e-5)
    sh = be_ref[...] - mu * a
    e = jnp.maximum(h_ref[...] * a + sh, 0.0)
    out_ref[...] = (jnp.sum(e * w_ref[...], axis=0, keepdims=True)
                    + b0_ref[...])


def _edge_mlp(SA, SB, ewT, M1c, M1b, g1, be1, M2wT, M2b, g2, be2,
              M3wT, M3b, g3, be3, M4w, M4b):
    f32 = jnp.float32
    h1, s1, q1 = pl.pallas_call(
        _p1_body,
        grid=(NB,),
        in_specs=[_eblk_spec(), _eblk_spec(),
                  pl.BlockSpec((1, BE), lambda i: (0, i)),
                  _col_spec(), _col_spec()],
        out_specs=[_eblk_spec(), _col_spec(), _col_spec()],
        out_shape=[jax.ShapeDtypeStruct((H, E), f32),
                   jax.ShapeDtypeStruct((H, 1), f32),
                   jax.ShapeDtypeStruct((H, 1), f32)],
    )(SA, SB, ewT, M1c, M1b)

    def mid(h, s, q, g, be, wT, b):
        return pl.pallas_call(
            _p23_body,
            grid=(NB,),
            in_specs=[_eblk_spec(), _col_spec(), _col_spec(), _col_spec(),
                      _col_spec(), pl.BlockSpec((H, H), lambda i: (0, 0)),
                      _col_spec()],
            out_specs=[_eblk_spec(), _col_spec(), _col_spec()],
            out_shape=[jax.ShapeDtypeStruct((H, E), f32),
                       jax.ShapeDtypeStruct((H, 1), f32),
                       jax.ShapeDtypeStruct((H, 1), f32)],
        )(h, s, q, g, be, wT, b)

    h2, s2, q2 = mid(h1, s1, q1, g1, be1, M2wT, M2b)
    h3, s3, q3 = mid(h2, s2, q2, g2, be2, M3wT, M3b)

    logitsT = pl.pallas_call(
        _p4_body,
        grid=(NB,),
        in_specs=[_eblk_spec(), _col_spec(), _col_spec(), _col_spec(),
                  _col_spec(), _col_spec(),
                  pl.BlockSpec((1, 1), lambda i: (0, 0))],
        out_specs=pl.BlockSpec((1, BE), lambda i: (0, i)),
        out_shape=jax.ShapeDtypeStruct((1, E), f32),
    )(h3, s3, q3, g3, be3, M4w, M4b)
    return logitsT


def kernel(x, edge_index, edge_weight, W1, b1, W2, b2, W3, b3,
           M1w, M1b, g1, be1, M2w, M2b, g2, be2, M3w, M3b, g3, be3,
           M4w, M4b):
    src, dst = edge_index[0], edge_index[1]
    ew = edge_weight

    xT = jnp.pad(x.T, ((0, 0), (0, NPAD - N)))
    W1T, W2T, W3T = W1.T, W2.T, W3.T
    M1aT = M1w[:H].T
    M1bT = M1w[H:2 * H].T

    degp = _deg_partials(dst, ew)
    t1T = _tk_matmul(W1T, xT)
    dinv, g1T = _tk_dinv(degp, t1T)

    agg1 = _sc_aggregate(g1T, src, dst, ew)
    t2T, g2T = _tk_mid(agg1, t1T, dinv, b1.reshape(H, 1), W2T)
    agg2 = _sc_aggregate(g2T, src, dst, ew)
    t3T, g3T = _tk_mid(agg2, t2T, dinv, b2.reshape(H, 1), W3T)
    agg3 = _sc_aggregate(g3T, src, dst, ew)
    nodeAT, nodeBT = _tk_node_tables(agg3, t3T, dinv, b3.reshape(H, 1),
                                     M1aT, M1bT)

    SA = _sc_gather_rows(nodeAT, src)
    SB = _sc_gather_rows(nodeBT, dst)

    logitsT = _edge_mlp(
        SA, SB, ew.reshape(1, E), M1w[2 * H].reshape(H, 1), M1b.reshape(H, 1),
        g1.reshape(H, 1), be1.reshape(H, 1), M2w.T, M2b.reshape(H, 1),
        g2.reshape(H, 1), be2.reshape(H, 1), M3w.T, M3b.reshape(H, 1),
        g3.reshape(H, 1), be3.reshape(H, 1), M4w.reshape(H, 1),
        M4b.reshape(1, 1))
    return logitsT.reshape(E)


# speedup vs baseline: 2.0638x; 1.2884x over previous
import dataclasses
import functools

import jax
import jax.numpy as jnp
from jax import lax
from jax.experimental import pallas as pl
from jax.experimental.pallas import tpu as pltpu
from jax.experimental.pallas import tpu_sc as plsc

N = 10000
NPAD = 10240
E = 320000
H = 256
NTILES = 32
FP = 4
NPASS = H // (NTILES * FP)
CH = 4000
CHD = 2000
EDGT = E // NTILES
BE = 3200
NB = E // BE

def _mesh():
    return plsc.VectorSubcoreMesh(core_axis_name="c", subcore_axis_name="s")


def _cp():
    cp = pltpu.CompilerParams()
    if "needs_layout_passes" in pltpu.CompilerParams.__dataclass_fields__:
        cp = dataclasses.replace(cp, needs_layout_passes=False)
    return cp


def _deg_partials(dst, ew):
    @functools.partial(
        pl.kernel, mesh=_mesh(), compiler_params=_cp(),
        out_type=jax.ShapeDtypeStruct((NTILES, NPAD), jnp.float32),
        scratch_types=[pltpu.VMEM((NPAD,), jnp.float32),
                       pltpu.VMEM((CHD,), jnp.int32),
                       pltpu.VMEM((CHD,), jnp.float32)])
    def k(dst_hbm, ew_hbm, out_hbm, accv, dv, wv):
        wid = lax.axis_index("s") * 2 + lax.axis_index("c")

        @pl.loop(0, NPAD, step=16)
        def _(j):
            accv[pl.ds(j, 16)] = jnp.zeros((16,), jnp.float32)

        base = wid * EDGT

        @pl.loop(0, EDGT, step=CHD)
        def _(c):
            pltpu.sync_copy(dst_hbm.at[pl.ds(base + c, CHD)], dv)
            pltpu.sync_copy(ew_hbm.at[pl.ds(base + c, CHD)], wv)

            @pl.loop(0, CHD, step=16)
            def _(g):
                plsc.addupdate_scatter(accv, [dv[pl.ds(g, 16)]],
                                       wv[pl.ds(g, 16)])

        pltpu.sync_copy(accv, out_hbm.at[wid])

    return k(dst, ew)


def _sc_aggregate(gT, src, dst, ew):
    @functools.partial(
        pl.kernel, mesh=_mesh(), compiler_params=_cp(),
        out_type=jax.ShapeDtypeStruct((H, NPAD), jnp.float32),
        scratch_types=[pltpu.VMEM((FP, NPAD), jnp.float32),
                       pltpu.VMEM((FP, NPAD), jnp.float32),
                       pltpu.VMEM((CH,), jnp.int32),
                       pltpu.VMEM((CH,), jnp.int32),
                       pltpu.VMEM((CH,), jnp.float32)])
    def k(gT_hbm, src_hbm, dst_hbm, ew_hbm, out_hbm, gv, accv, sv, dv, wv):
        wid = lax.axis_index("s") * 2 + lax.axis_index("c")
        for p in range(NPASS):
            fbase = (p * NTILES + wid) * FP
            pltpu.sync_copy(gT_hbm.at[pl.ds(fbase, FP)], gv)
            for f in range(FP):
                @pl.loop(0, NPAD, step=16)
                def _(j, f=f):
                    accv[f, pl.ds(j, 16)] = jnp.zeros((16,), jnp.float32)

            @pl.loop(0, E, step=CH)
            def _(c):
                pltpu.sync_copy(src_hbm.at[pl.ds(c, CH)], sv)
                pltpu.sync_copy(dst_hbm.at[pl.ds(c, CH)], dv)
                pltpu.sync_copy(ew_hbm.at[pl.ds(c, CH)], wv)

                @pl.loop(0, CH, step=32)
                def _(g):
                    for u in range(2):
                        s16 = sv[pl.ds(g + 16 * u, 16)]
                        d16 = dv[pl.ds(g + 16 * u, 16)]
                        w16 = wv[pl.ds(g + 16 * u, 16)]
                        for f in range(FP):
                            f16 = jnp.full((16,), f, jnp.int32)
                            vals = plsc.load_gather(gv, [f16, s16]) * w16
                            plsc.addupdate_scatter(accv, [f16, d16], vals)

            pltpu.sync_copy(accv, out_hbm.at[pl.ds(fbase, FP)])

    return k(gT, src, dst, ew)


def _sc_edge_rows(t_nm, idx):
    KE = 80

    @functools.partial(
        pl.kernel, mesh=_mesh(), compiler_params=_cp(),
        out_type=jax.ShapeDtypeStruct((E, H), jnp.float32),
        scratch_types=[pltpu.VMEM((KE,), jnp.int32),
                       pltpu.VMEM((KE, H), jnp.float32)])
    def k(t_hbm, i_hbm, out_hbm, iv, buf):
        wid = lax.axis_index("s") * 2 + lax.axis_index("c")
        base = wid * (E // NTILES)

        @pl.loop(0, E // NTILES, step=KE)
        def _(c):
            pltpu.sync_copy(i_hbm.at[pl.ds(base + c, KE)], iv)
            pltpu.sync_copy(t_hbm.at[iv], buf)
            pltpu.sync_copy(buf, out_hbm.at[pl.ds(base + c, KE)])

    return k(t_nm, idx)


def _bdot_t(a, b):
    return lax.dot_general(a.astype(jnp.bfloat16), b.astype(jnp.bfloat16),
                           (((0,), (0,)), ((), ())),
                           preferred_element_type=jnp.float32)


def _bdot(a, b):
    return jnp.dot(a.astype(jnp.bfloat16), b.astype(jnp.bfloat16),
                   preferred_element_type=jnp.float32)


def _tk_matmul(wT, xT):
    def body(w_ref, x_ref, o_ref):
        o_ref[...] = _bdot(w_ref[...], x_ref[...])
    return pl.pallas_call(
        body, out_shape=jax.ShapeDtypeStruct((wT.shape[0], xT.shape[1]),
                                             jnp.float32))(wT, xT)


def _tk_dinv(degp, t1T):
    def body(dp_ref, t_ref, dinv_ref, g_ref):
        deg = jnp.sum(dp_ref[...], axis=0, keepdims=True) + 1.0
        dinv = jax.lax.rsqrt(deg)
        dinv_ref[...] = dinv
        g_ref[...] = t_ref[...] * dinv
    return pl.pallas_call(
        body, out_shape=[jax.ShapeDtypeStruct((1, NPAD), jnp.float32),
                         jax.ShapeDtypeStruct((H, NPAD), jnp.float32)])(degp, t1T)


NCB = 2048
NCBLK = NPAD // NCB


def _nblk():
    return pl.BlockSpec((H, NCB), lambda i: (0, i))


def _tk_mid(agg, tT, dinv, b, wT):
    def body(a_ref, t_ref, dinv_ref, b_ref, w_ref, tn_ref, gn_ref):
        dinv = dinv_ref[...]
        h = jnp.maximum(dinv * a_ref[...] + dinv * dinv * t_ref[...]
                        + b_ref[...], 0.0)
        tn = _bdot(w_ref[...], h)
        tn_ref[...] = tn
        gn_ref[...] = tn * dinv
    return pl.pallas_call(
        body, grid=(NCBLK,),
        in_specs=[_nblk(), _nblk(),
                  pl.BlockSpec((1, NCB), lambda i: (0, i)),
                  pl.BlockSpec((H, 1), lambda i: (0, 0)),
                  pl.BlockSpec((H, H), lambda i: (0, 0))],
        out_specs=[_nblk(), _nblk()],
        out_shape=[jax.ShapeDtypeStruct((H, NPAD), jnp.float32),
                   jax.ShapeDtypeStruct((H, NPAD), jnp.float32)])(
        agg, tT, dinv, b, wT)


def _tk_node_tables(agg, tT, dinv, b, waT, wbT):
    def body(a_ref, t_ref, dinv_ref, b_ref, wa_ref, wb_ref, oa_ref, ob_ref):
        dinv = dinv_ref[...]
        node = dinv * a_ref[...] + dinv * dinv * t_ref[...] + b_ref[...]
        oa_ref[...] = _bdot_t(node, wa_ref[...])
        ob_ref[...] = _bdot_t(node, wb_ref[...])
    return pl.pallas_call(
        body, grid=(NCBLK,),
        in_specs=[_nblk(), _nblk(),
                  pl.BlockSpec((1, NCB), lambda i: (0, i)),
                  pl.BlockSpec((H, 1), lambda i: (0, 0)),
                  pl.BlockSpec((H, H), lambda i: (0, 0)),
                  pl.BlockSpec((H, H), lambda i: (0, 0))],
        out_specs=[pl.BlockSpec((NCB, H), lambda i: (i, 0)),
                   pl.BlockSpec((NCB, H), lambda i: (i, 0))],
        out_shape=[jax.ShapeDtypeStruct((NPAD, H), jnp.float32),
                   jax.ShapeDtypeStruct((NPAD, H), jnp.float32)])(
        agg, tT, dinv, b, waT, wbT)


BE2 = 4000
NB2 = E // BE2


def _row_spec():
    return pl.BlockSpec((1, H), lambda i: (0, 0))


def _eblk():
    return pl.BlockSpec((BE2, H), lambda i: (i, 0))


def _p1_body(sa_ref, sb_ref, ew_ref, c_ref, b_ref, h_ref, s_ref, q_ref):
    i = pl.program_id(0)
    ew16 = ew_ref[...].astype(jnp.bfloat16).astype(jnp.float32)
    c16 = c_ref[...].astype(jnp.bfloat16).astype(jnp.float32)
    h = sa_ref[...] + sb_ref[...] + ew16 * c16 + b_ref[...]
    h_ref[...] = h

    @pl.when(i == 0)
    def _():
        s_ref[...] = jnp.zeros_like(s_ref)
        q_ref[...] = jnp.zeros_like(q_ref)

    s_ref[...] += jnp.sum(h, axis=0, keepdims=True)
    q_ref[...] += jnp.sum(h * h, axis=0, keepdims=True)


def _p23_body(h_ref, s_in, q_in, g_ref, be_ref, w_ref, b_ref,
              out_ref, s_ref, q_ref):
    i = pl.program_id(0)
    mu = s_in[...] / E
    var = q_in[...] / E - mu * mu
    a = g_ref[...] * jax.lax.rsqrt(var + 1e-5)
    sh = be_ref[...] - mu * a
    e = jnp.maximum(h_ref[...] * a + sh, 0.0)
    h2 = _bdot(e, w_ref[...]) + b_ref[...]
    out_ref[...] = h2

    @pl.when(i == 0)
    def _():
        s_ref[...] = jnp.zeros_like(s_ref)
        q_ref[...] = jnp.zeros_like(q_ref)

    s_ref[...] += jnp.sum(h2, axis=0, keepdims=True)
    q_ref[...] += jnp.sum(h2 * h2, axis=0, keepdims=True)


def _p4_body(h_ref, s_in, q_in, g_ref, be_ref, w_ref, b0_ref, out_ref):
    mu = s_in[...] / E
    var = q_in[...] / E - mu * mu
    a = g_ref[...] * jax.lax.rsqrt(var + 1e-5)
    sh = be_ref[...] - mu * a
    e = jnp.maximum(h_ref[...] * a + sh, 0.0)
    e16 = e.astype(jnp.bfloat16).astype(jnp.float32)
    w16 = w_ref[...].astype(jnp.bfloat16).astype(jnp.float32)
    out_ref[...] = (jnp.sum(e16 * w16, axis=1, keepdims=True)
                    + b0_ref[...])


def _edge_mlp(SA, SB, ew_col, M1c, M1b, g1, be1, M2w, M2b, g2, be2,
              M3w, M3b, g3, be3, M4w, M4b):
    f32 = jnp.float32
    h1, s1, q1 = pl.pallas_call(
        _p1_body,
        grid=(NB2,),
        in_specs=[_eblk(), _eblk(),
                  pl.BlockSpec((BE2, 1), lambda i: (i, 0)),
                  _row_spec(), _row_spec()],
        out_specs=[_eblk(), _row_spec(), _row_spec()],
        out_shape=[jax.ShapeDtypeStruct((E, H), f32),
                   jax.ShapeDtypeStruct((1, H), f32),
                   jax.ShapeDtypeStruct((1, H), f32)],
    )(SA, SB, ew_col, M1c, M1b)

    def mid(h, s, q, g, be, w, b):
        return pl.pallas_call(
            _p23_body,
            grid=(NB2,),
            in_specs=[_eblk(), _row_spec(), _row_spec(), _row_spec(),
                      _row_spec(), pl.BlockSpec((H, H), lambda i: (0, 0)),
                      _row_spec()],
            out_specs=[_eblk(), _row_spec(), _row_spec()],
            out_shape=[jax.ShapeDtypeStruct((E, H), f32),
                       jax.ShapeDtypeStruct((1, H), f32),
                       jax.ShapeDtypeStruct((1, H), f32)],
        )(h, s, q, g, be, w, b)

    h2, s2, q2 = mid(h1, s1, q1, g1, be1, M2w, M2b)
    h3, s3, q3 = mid(h2, s2, q2, g2, be2, M3w, M3b)

    logits = pl.pallas_call(
        _p4_body,
        grid=(NB2,),
        in_specs=[_eblk(), _row_spec(), _row_spec(), _row_spec(),
                  _row_spec(), _row_spec(),
                  pl.BlockSpec((1, 1), lambda i: (0, 0))],
        out_specs=pl.BlockSpec((BE2, 1), lambda i: (i, 0)),
        out_shape=jax.ShapeDtypeStruct((E, 1), f32),
    )(h3, s3, q3, g3, be3, M4w, M4b)
    return logits


def kernel(x, edge_index, edge_weight, W1, b1, W2, b2, W3, b3,
           M1w, M1b, g1, be1, M2w, M2b, g2, be2, M3w, M3b, g3, be3,
           M4w, M4b):
    src, dst = edge_index[0], edge_index[1]
    ew = edge_weight

    xT = jnp.pad(x.T, ((0, 0), (0, NPAD - N)))
    W1T, W2T, W3T = W1.T, W2.T, W3.T

    degp = _deg_partials(dst, ew)
    t1T = _tk_matmul(W1T, xT)
    dinv, g1T = _tk_dinv(degp, t1T)

    agg1 = _sc_aggregate(g1T, src, dst, ew)
    t2T, g2T = _tk_mid(agg1, t1T, dinv, b1.reshape(H, 1), W2T)
    agg2 = _sc_aggregate(g2T, src, dst, ew)
    t3T, g3T = _tk_mid(agg2, t2T, dinv, b2.reshape(H, 1), W3T)
    agg3 = _sc_aggregate(g3T, src, dst, ew)
    nodeA, nodeB = _tk_node_tables(agg3, t3T, dinv, b3.reshape(H, 1),
                                   M1w[:H], M1w[H:2 * H])

    SA = _sc_edge_rows(nodeA, src)
    SB = _sc_edge_rows(nodeB, dst)

    logits = _edge_mlp(
        SA, SB, ew.reshape(E, 1), M1w[2 * H].reshape(1, H), M1b.reshape(1, H),
        g1.reshape(1, H), be1.reshape(1, H), M2w, M2b.reshape(1, H),
        g2.reshape(1, H), be2.reshape(1, H), M3w, M3b.reshape(1, H),
        g3.reshape(1, H), be3.reshape(1, H), M4w.reshape(1, H),
        M4b.reshape(1, 1))
    return logits.reshape(E)


# speedup vs baseline: 2.4492x; 1.1868x over previous
import dataclasses
import functools

import jax
import jax.numpy as jnp
from jax import lax
from jax.experimental import pallas as pl
from jax.experimental.pallas import tpu as pltpu
from jax.experimental.pallas import tpu_sc as plsc

N = 10000
NPAD = 10240
E = 320000
H = 256
NTILES = 32
FP = 4
NPASS = H // (NTILES * FP)
CH = 4000
CHD = 2000
EDGT = E // NTILES
BE = 3200
NB = E // BE

def _mesh():
    return plsc.VectorSubcoreMesh(core_axis_name="c", subcore_axis_name="s")


def _cp():
    cp = pltpu.CompilerParams()
    if "needs_layout_passes" in pltpu.CompilerParams.__dataclass_fields__:
        cp = dataclasses.replace(cp, needs_layout_passes=False)
    return cp


def _deg_partials(dst, ew):
    @functools.partial(
        pl.kernel, mesh=_mesh(), compiler_params=_cp(),
        out_type=jax.ShapeDtypeStruct((NTILES, NPAD), jnp.float32),
        scratch_types=[pltpu.VMEM((NPAD,), jnp.float32),
                       pltpu.VMEM((CHD,), jnp.int32),
                       pltpu.VMEM((CHD,), jnp.float32)])
    def k(dst_hbm, ew_hbm, out_hbm, accv, dv, wv):
        wid = lax.axis_index("s") * 2 + lax.axis_index("c")

        @pl.loop(0, NPAD, step=16)
        def _(j):
            accv[pl.ds(j, 16)] = jnp.zeros((16,), jnp.float32)

        base = wid * EDGT

        @pl.loop(0, EDGT, step=CHD)
        def _(c):
            pltpu.sync_copy(dst_hbm.at[pl.ds(base + c, CHD)], dv)
            pltpu.sync_copy(ew_hbm.at[pl.ds(base + c, CHD)], wv)

            @pl.loop(0, CHD, step=16)
            def _(g):
                plsc.addupdate_scatter(accv, [dv[pl.ds(g, 16)]],
                                       wv[pl.ds(g, 16)])

        pltpu.sync_copy(accv, out_hbm.at[wid])

    return k(dst, ew)


def _sc_aggregate(gT, src, dst, ew):
    NCHK = E // CH

    @functools.partial(
        pl.kernel, mesh=_mesh(), compiler_params=_cp(),
        out_type=jax.ShapeDtypeStruct((H, NPAD), jnp.float32),
        scratch_types=[pltpu.VMEM((FP, NPAD), jnp.float32),
                       pltpu.VMEM((FP, NPAD), jnp.float32),
                       pltpu.VMEM((CH,), jnp.int32),
                       pltpu.VMEM((CH,), jnp.int32),
                       pltpu.VMEM((CH,), jnp.int32),
                       pltpu.VMEM((CH,), jnp.int32),
                       pltpu.VMEM((CH,), jnp.float32),
                       pltpu.VMEM((CH,), jnp.float32)]
        + [pltpu.SemaphoreType.DMA] * 6)
    def k(gT_hbm, src_hbm, dst_hbm, ew_hbm, out_hbm, gv, accv, sv0, sv1,
          dv0, dv1, wv0, wv1, ss0, ss1, sd0, sd1, sw0, sw1):
        wid = lax.axis_index("s") * 2 + lax.axis_index("c")
        svs = (sv0, sv1)
        dvs = (dv0, dv1)
        wvs = (wv0, wv1)
        ssem = (ss0, ss1)
        dsem = (sd0, sd1)
        wsem = (sw0, sw1)

        def start(c, b):
            pltpu.make_async_copy(src_hbm.at[pl.ds(c, CH)], svs[b],
                                  ssem[b]).start()
            pltpu.make_async_copy(dst_hbm.at[pl.ds(c, CH)], dvs[b],
                                  dsem[b]).start()
            pltpu.make_async_copy(ew_hbm.at[pl.ds(c, CH)], wvs[b],
                                  wsem[b]).start()

        def wait(c, b):
            pltpu.make_async_copy(src_hbm.at[pl.ds(c, CH)], svs[b],
                                  ssem[b]).wait()
            pltpu.make_async_copy(dst_hbm.at[pl.ds(c, CH)], dvs[b],
                                  dsem[b]).wait()
            pltpu.make_async_copy(ew_hbm.at[pl.ds(c, CH)], wvs[b],
                                  wsem[b]).wait()

        for p in range(NPASS):
            fbase = (p * NTILES + wid) * FP
            pltpu.sync_copy(gT_hbm.at[pl.ds(fbase, FP)], gv)
            for f in range(FP):
                @pl.loop(0, NPAD, step=16)
                def _(j, f=f):
                    accv[f, pl.ds(j, 16)] = jnp.zeros((16,), jnp.float32)

            start(0, 0)
            start(CH, 1)

            @pl.loop(0, E, step=2 * CH)
            def _(cc):
                for b in range(2):
                    c = cc + b * CH
                    wait(c, b)

                    @pl.loop(0, CH, step=32)
                    def _(g, b=b):
                        for u in range(2):
                            s16 = svs[b][pl.ds(g + 16 * u, 16)]
                            d16 = dvs[b][pl.ds(g + 16 * u, 16)]
                            w16 = wvs[b][pl.ds(g + 16 * u, 16)]
                            for f in range(FP):
                                f16 = jnp.full((16,), f, jnp.int32)
                                vals = plsc.load_gather(gv, [f16, s16]) * w16
                                plsc.addupdate_scatter(accv, [f16, d16], vals)

                    @pl.when(c + 2 * CH < E)
                    def _(c=c, b=b):
                        start(c + 2 * CH, b)

            pltpu.sync_copy(accv, out_hbm.at[pl.ds(fbase, FP)])

    return k(gT, src, dst, ew)


def _sc_edge_rows(t_nm, idx):
    KE = 80

    @functools.partial(
        pl.kernel, mesh=_mesh(), compiler_params=_cp(),
        out_type=jax.ShapeDtypeStruct((E, H), jnp.float32),
        scratch_types=[pltpu.VMEM((KE,), jnp.int32),
                       pltpu.VMEM((KE, H), jnp.float32)])
    def k(t_hbm, i_hbm, out_hbm, iv, buf):
        wid = lax.axis_index("s") * 2 + lax.axis_index("c")
        base = wid * (E // NTILES)

        @pl.loop(0, E // NTILES, step=KE)
        def _(c):
            pltpu.sync_copy(i_hbm.at[pl.ds(base + c, KE)], iv)
            pltpu.sync_copy(t_hbm.at[iv], buf)
            pltpu.sync_copy(buf, out_hbm.at[pl.ds(base + c, KE)])

    return k(t_nm, idx)


def _bdot_t(a, b):
    return lax.dot_general(a.astype(jnp.bfloat16), b.astype(jnp.bfloat16),
                           (((0,), (0,)), ((), ())),
                           preferred_element_type=jnp.float32)


def _bdot(a, b):
    return jnp.dot(a.astype(jnp.bfloat16), b.astype(jnp.bfloat16),
                   preferred_element_type=jnp.float32)


def _tk_matmul(wT, xT):
    def body(w_ref, x_ref, o_ref):
        o_ref[...] = _bdot(w_ref[...], x_ref[...])
    return pl.pallas_call(
        body, out_shape=jax.ShapeDtypeStruct((wT.shape[0], xT.shape[1]),
                                             jnp.float32))(wT, xT)


def _tk_dinv(degp, t1T):
    def body(dp_ref, t_ref, dinv_ref, g_ref):
        deg = jnp.sum(dp_ref[...], axis=0, keepdims=True) + 1.0
        dinv = jax.lax.rsqrt(deg)
        dinv_ref[...] = dinv
        g_ref[...] = t_ref[...] * dinv
    return pl.pallas_call(
        body, out_shape=[jax.ShapeDtypeStruct((1, NPAD), jnp.float32),
                         jax.ShapeDtypeStruct((H, NPAD), jnp.float32)])(degp, t1T)


NCB = 2048
NCBLK = NPAD // NCB


def _nblk():
    return pl.BlockSpec((H, NCB), lambda i: (0, i))


def _tk_mid(agg, tT, dinv, b, wT):
    def body(a_ref, t_ref, dinv_ref, b_ref, w_ref, tn_ref, gn_ref):
        dinv = dinv_ref[...]
        h = jnp.maximum(dinv * a_ref[...] + dinv * dinv * t_ref[...]
                        + b_ref[...], 0.0)
        tn = _bdot(w_ref[...], h)
        tn_ref[...] = tn
        gn_ref[...] = tn * dinv
    return pl.pallas_call(
        body, grid=(NCBLK,),
        in_specs=[_nblk(), _nblk(),
                  pl.BlockSpec((1, NCB), lambda i: (0, i)),
                  pl.BlockSpec((H, 1), lambda i: (0, 0)),
                  pl.BlockSpec((H, H), lambda i: (0, 0))],
        out_specs=[_nblk(), _nblk()],
        out_shape=[jax.ShapeDtypeStruct((H, NPAD), jnp.float32),
                   jax.ShapeDtypeStruct((H, NPAD), jnp.float32)])(
        agg, tT, dinv, b, wT)


def _tk_node_tables(agg, tT, dinv, b, waT, wbT):
    def body(a_ref, t_ref, dinv_ref, b_ref, wa_ref, wb_ref, oa_ref, ob_ref):
        dinv = dinv_ref[...]
        node = dinv * a_ref[...] + dinv * dinv * t_ref[...] + b_ref[...]
        oa_ref[...] = _bdot_t(node, wa_ref[...])
        ob_ref[...] = _bdot_t(node, wb_ref[...])
    return pl.pallas_call(
        body, grid=(NCBLK,),
        in_specs=[_nblk(), _nblk(),
                  pl.BlockSpec((1, NCB), lambda i: (0, i)),
                  pl.BlockSpec((H, 1), lambda i: (0, 0)),
                  pl.BlockSpec((H, H), lambda i: (0, 0)),
                  pl.BlockSpec((H, H), lambda i: (0, 0))],
        out_specs=[pl.BlockSpec((NCB, H), lambda i: (i, 0)),
                   pl.BlockSpec((NCB, H), lambda i: (i, 0))],
        out_shape=[jax.ShapeDtypeStruct((NPAD, H), jnp.float32),
                   jax.ShapeDtypeStruct((NPAD, H), jnp.float32)])(
        agg, tT, dinv, b, waT, wbT)


BE2 = 4000
NB2 = E // BE2


def _row_spec():
    return pl.BlockSpec((1, H), lambda i: (0, 0))


def _eblk():
    return pl.BlockSpec((BE2, H), lambda i: (i, 0))


def _p1_body(sa_ref, sb_ref, ew_ref, c_ref, b_ref, h_ref, s_ref, q_ref):
    i = pl.program_id(0)
    ew16 = ew_ref[...].astype(jnp.bfloat16).astype(jnp.float32)
    c16 = c_ref[...].astype(jnp.bfloat16).astype(jnp.float32)
    h = sa_ref[...] + sb_ref[...] + ew16 * c16 + b_ref[...]
    h_ref[...] = h

    @pl.when(i == 0)
    def _():
        s_ref[...] = jnp.zeros_like(s_ref)
        q_ref[...] = jnp.zeros_like(q_ref)

    s_ref[...] += jnp.sum(h, axis=0, keepdims=True)
    q_ref[...] += jnp.sum(h * h, axis=0, keepdims=True)


def _p23_body(h_ref, s_in, q_in, g_ref, be_ref, w_ref, b_ref,
              out_ref, s_ref, q_ref):
    i = pl.program_id(0)
    mu = s_in[...] / E
    var = q_in[...] / E - mu * mu
    a = g_ref[...] * jax.lax.rsqrt(var + 1e-5)
    sh = be_ref[...] - mu * a
    e = jnp.maximum(h_ref[...] * a + sh, 0.0)
    h2 = _bdot(e, w_ref[...]) + b_ref[...]
    out_ref[...] = h2

    @pl.when(i == 0)
    def _():
        s_ref[...] = jnp.zeros_like(s_ref)
        q_ref[...] = jnp.zeros_like(q_ref)

    s_ref[...] += jnp.sum(h2, axis=0, keepdims=True)
    q_ref[...] += jnp.sum(h2 * h2, axis=0, keepdims=True)


def _p4_body(h_ref, s_in, q_in, g_ref, be_ref, w_ref, b0_ref, out_ref):
    mu = s_in[...] / E
    var = q_in[...] / E - mu * mu
    a = g_ref[...] * jax.lax.rsqrt(var + 1e-5)
    sh = be_ref[...] - mu * a
    e = jnp.maximum(h_ref[...] * a + sh, 0.0)
    e16 = e.astype(jnp.bfloat16).astype(jnp.float32)
    w16 = w_ref[...].astype(jnp.bfloat16).astype(jnp.float32)
    out_ref[...] = (jnp.sum(e16 * w16, axis=1, keepdims=True)
                    + b0_ref[...])


def _edge_mlp(SA, SB, ew_col, M1c, M1b, g1, be1, M2w, M2b, g2, be2,
              M3w, M3b, g3, be3, M4w, M4b):
    f32 = jnp.float32
    h1, s1, q1 = pl.pallas_call(
        _p1_body,
        grid=(NB2,),
        in_specs=[_eblk(), _eblk(),
                  pl.BlockSpec((BE2, 1), lambda i: (i, 0)),
                  _row_spec(), _row_spec()],
        out_specs=[_eblk(), _row_spec(), _row_spec()],
        out_shape=[jax.ShapeDtypeStruct((E, H), f32),
                   jax.ShapeDtypeStruct((1, H), f32),
                   jax.ShapeDtypeStruct((1, H), f32)],
    )(SA, SB, ew_col, M1c, M1b)

    def mid(h, s, q, g, be, w, b):
        return pl.pallas_call(
            _p23_body,
            grid=(NB2,),
            in_specs=[_eblk(), _row_spec(), _row_spec(), _row_spec(),
                      _row_spec(), pl.BlockSpec((H, H), lambda i: (0, 0)),
                      _row_spec()],
            out_specs=[_eblk(), _row_spec(), _row_spec()],
            out_shape=[jax.ShapeDtypeStruct((E, H), f32),
                       jax.ShapeDtypeStruct((1, H), f32),
                       jax.ShapeDtypeStruct((1, H), f32)],
        )(h, s, q, g, be, w, b)

    h2, s2, q2 = mid(h1, s1, q1, g1, be1, M2w, M2b)
    h3, s3, q3 = mid(h2, s2, q2, g2, be2, M3w, M3b)

    logits = pl.pallas_call(
        _p4_body,
        grid=(NB2,),
        in_specs=[_eblk(), _row_spec(), _row_spec(), _row_spec(),
                  _row_spec(), _row_spec(),
                  pl.BlockSpec((1, 1), lambda i: (0, 0))],
        out_specs=pl.BlockSpec((BE2, 1), lambda i: (i, 0)),
        out_shape=jax.ShapeDtypeStruct((E, 1), f32),
    )(h3, s3, q3, g3, be3, M4w, M4b)
    return logits


def kernel(x, edge_index, edge_weight, W1, b1, W2, b2, W3, b3,
           M1w, M1b, g1, be1, M2w, M2b, g2, be2, M3w, M3b, g3, be3,
           M4w, M4b):
    src, dst = edge_index[0], edge_index[1]
    ew = edge_weight

    xT = jnp.pad(x.T, ((0, 0), (0, NPAD - N)))
    W1T, W2T, W3T = W1.T, W2.T, W3.T

    degp = _deg_partials(dst, ew)
    t1T = _tk_matmul(W1T, xT)
    dinv, g1T = _tk_dinv(degp, t1T)

    agg1 = _sc_aggregate(g1T, src, dst, ew)
    t2T, g2T = _tk_mid(agg1, t1T, dinv, b1.reshape(H, 1), W2T)
    agg2 = _sc_aggregate(g2T, src, dst, ew)
    t3T, g3T = _tk_mid(agg2, t2T, dinv, b2.reshape(H, 1), W3T)
    agg3 = _sc_aggregate(g3T, src, dst, ew)
    nodeA, nodeB = _tk_node_tables(agg3, t3T, dinv, b3.reshape(H, 1),
                                   M1w[:H], M1w[H:2 * H])

    SA = _sc_edge_rows(nodeA, src)
    SB = _sc_edge_rows(nodeB, dst)

    logits = _edge_mlp(
        SA, SB, ew.reshape(E, 1), M1w[2 * H].reshape(1, H), M1b.reshape(1, H),
        g1.reshape(1, H), be1.reshape(1, H), M2w, M2b.reshape(1, H),
        g2.reshape(1, H), be2.reshape(1, H), M3w, M3b.reshape(1, H),
        g3.reshape(1, H), be3.reshape(1, H), M4w.reshape(1, H),
        M4b.reshape(1, 1))
    return logits.reshape(E)


# speedup vs baseline: 2.4729x; 1.0097x over previous
import dataclasses
import functools

import jax
import jax.numpy as jnp
from jax import lax
from jax.experimental import pallas as pl
from jax.experimental.pallas import tpu as pltpu
from jax.experimental.pallas import tpu_sc as plsc

N = 10000
NPAD = 10240
E = 320000
H = 256
NTILES = 32
FP = 4
NPASS = H // (NTILES * FP)
CH = 4000
CHD = 2000
EDGT = E // NTILES
BE = 3200
NB = E // BE

def _mesh():
    return plsc.VectorSubcoreMesh(core_axis_name="c", subcore_axis_name="s")


def _cp():
    cp = pltpu.CompilerParams()
    if "needs_layout_passes" in pltpu.CompilerParams.__dataclass_fields__:
        cp = dataclasses.replace(cp, needs_layout_passes=False)
    return cp


def _deg_partials(dst, ew):
    @functools.partial(
        pl.kernel, mesh=_mesh(), compiler_params=_cp(),
        out_type=jax.ShapeDtypeStruct((NTILES, NPAD), jnp.float32),
        scratch_types=[pltpu.VMEM((NPAD,), jnp.float32),
                       pltpu.VMEM((CHD,), jnp.int32),
                       pltpu.VMEM((CHD,), jnp.float32)])
    def k(dst_hbm, ew_hbm, out_hbm, accv, dv, wv):
        wid = lax.axis_index("s") * 2 + lax.axis_index("c")

        @pl.loop(0, NPAD, step=16)
        def _(j):
            accv[pl.ds(j, 16)] = jnp.zeros((16,), jnp.float32)

        base = wid * EDGT

        @pl.loop(0, EDGT, step=CHD)
        def _(c):
            pltpu.sync_copy(dst_hbm.at[pl.ds(base + c, CHD)], dv)
            pltpu.sync_copy(ew_hbm.at[pl.ds(base + c, CHD)], wv)

            @pl.loop(0, CHD, step=16)
            def _(g):
                plsc.addupdate_scatter(accv, [dv[pl.ds(g, 16)]],
                                       wv[pl.ds(g, 16)])

        pltpu.sync_copy(accv, out_hbm.at[wid])

    return k(dst, ew)


def _sc_aggregate(gT, src, dst, ew):
    NCHK = E // CH

    @functools.partial(
        pl.kernel, mesh=_mesh(), compiler_params=_cp(),
        out_type=jax.ShapeDtypeStruct((H, NPAD), jnp.float32),
        scratch_types=[pltpu.VMEM((FP, NPAD), jnp.float32),
                       pltpu.VMEM((FP, NPAD), jnp.float32),
                       pltpu.VMEM((CH,), jnp.int32),
                       pltpu.VMEM((CH,), jnp.int32),
                       pltpu.VMEM((CH,), jnp.int32),
                       pltpu.VMEM((CH,), jnp.int32),
                       pltpu.VMEM((CH,), jnp.float32),
                       pltpu.VMEM((CH,), jnp.float32)]
        + [pltpu.SemaphoreType.DMA] * 6)
    def k(gT_hbm, src_hbm, dst_hbm, ew_hbm, out_hbm, gv, accv, sv0, sv1,
          dv0, dv1, wv0, wv1, ss0, ss1, sd0, sd1, sw0, sw1):
        wid = lax.axis_index("s") * 2 + lax.axis_index("c")
        svs = (sv0, sv1)
        dvs = (dv0, dv1)
        wvs = (wv0, wv1)
        ssem = (ss0, ss1)
        dsem = (sd0, sd1)
        wsem = (sw0, sw1)

        def start(c, b):
            pltpu.make_async_copy(src_hbm.at[pl.ds(c, CH)], svs[b],
                                  ssem[b]).start()
            pltpu.make_async_copy(dst_hbm.at[pl.ds(c, CH)], dvs[b],
                                  dsem[b]).start()
            pltpu.make_async_copy(ew_hbm.at[pl.ds(c, CH)], wvs[b],
                                  wsem[b]).start()

        def wait(c, b):
            pltpu.make_async_copy(src_hbm.at[pl.ds(c, CH)], svs[b],
                                  ssem[b]).wait()
            pltpu.make_async_copy(dst_hbm.at[pl.ds(c, CH)], dvs[b],
                                  dsem[b]).wait()
            pltpu.make_async_copy(ew_hbm.at[pl.ds(c, CH)], wvs[b],
                                  wsem[b]).wait()

        for p in range(NPASS):
            fbase = (p * NTILES + wid) * FP
            pltpu.sync_copy(gT_hbm.at[pl.ds(fbase, FP)], gv)
            for f in range(FP):
                @pl.loop(0, NPAD, step=16)
                def _(j, f=f):
                    accv[f, pl.ds(j, 16)] = jnp.zeros((16,), jnp.float32)

            start(0, 0)
            start(CH, 1)

            @pl.loop(0, E, step=2 * CH)
            def _(cc):
                for b in range(2):
                    c = cc + b * CH
                    wait(c, b)

                    @pl.loop(0, CH, step=80)
                    def _(g, b=b):
                        for u in range(5):
                            s16 = svs[b][pl.ds(g + 16 * u, 16)]
                            d16 = dvs[b][pl.ds(g + 16 * u, 16)]
                            w16 = wvs[b][pl.ds(g + 16 * u, 16)]
                            for f in range(FP):
                                f16 = jnp.full((16,), f, jnp.int32)
                                vals = plsc.load_gather(gv, [f16, s16]) * w16
                                plsc.addupdate_scatter(accv, [f16, d16], vals)

                    @pl.when(c + 2 * CH < E)
                    def _(c=c, b=b):
                        start(c + 2 * CH, b)

            pltpu.sync_copy(accv, out_hbm.at[pl.ds(fbase, FP)])

    return k(gT, src, dst, ew)


def _sc_edge_rows(t_nm, idx):
    KE = 40
    ET = E // NTILES

    @functools.partial(
        pl.kernel, mesh=_mesh(), compiler_params=_cp(),
        out_type=jax.ShapeDtypeStruct((E, H), jnp.float32),
        scratch_types=[pltpu.VMEM((KE,), jnp.int32),
                       pltpu.VMEM((KE,), jnp.int32),
                       pltpu.VMEM((KE, H), jnp.float32),
                       pltpu.VMEM((KE, H), jnp.float32)]
        + [pltpu.SemaphoreType.DMA] * 4)
    def k(t_hbm, i_hbm, out_hbm, iv0, iv1, ov0, ov1, si0, si1, so0, so1):
        wid = lax.axis_index("s") * 2 + lax.axis_index("c")
        base = wid * ET
        ivs = (iv0, iv1)
        ovs = (ov0, ov1)
        isem = (si0, si1)
        osem = (so0, so1)

        def istart(c, b):
            pltpu.make_async_copy(i_hbm.at[pl.ds(base + c, KE)], ivs[b],
                                  isem[b]).start()

        def iwait(c, b):
            pltpu.make_async_copy(i_hbm.at[pl.ds(base + c, KE)], ivs[b],
                                  isem[b]).wait()

        def ostart(c, b):
            pltpu.make_async_copy(ovs[b], out_hbm.at[pl.ds(base + c, KE)],
                                  osem[b]).start()

        def owait(c, b):
            pltpu.make_async_copy(ovs[b], out_hbm.at[pl.ds(base + c, KE)],
                                  osem[b]).wait()

        istart(0, 0)
        istart(KE, 1)

        @pl.loop(0, ET, step=2 * KE)
        def _(cc):
            for b in range(2):
                c = cc + b * KE
                iwait(c, b)

                @pl.when(c >= 2 * KE)
                def _(c=c, b=b):
                    owait(c - 2 * KE, b)

                pltpu.sync_copy(t_hbm.at[ivs[b]], ovs[b])
                ostart(c, b)

                @pl.when(c + 2 * KE < ET)
                def _(c=c, b=b):
                    istart(c + 2 * KE, b)

        owait(ET - 2 * KE, 0)
        owait(ET - KE, 1)

    return k(t_nm, idx)


def _bdot_t(a, b):
    return lax.dot_general(a.astype(jnp.bfloat16), b.astype(jnp.bfloat16),
                           (((0,), (0,)), ((), ())),
                           preferred_element_type=jnp.float32)


def _bdot(a, b):
    return jnp.dot(a.astype(jnp.bfloat16), b.astype(jnp.bfloat16),
                   preferred_element_type=jnp.float32)


def _tk_matmul(wT, xT):
    def body(w_ref, x_ref, o_ref):
        o_ref[...] = _bdot(w_ref[...], x_ref[...])
    return pl.pallas_call(
        body, out_shape=jax.ShapeDtypeStruct((wT.shape[0], xT.shape[1]),
                                             jnp.float32))(wT, xT)


def _tk_dinv(degp, t1T):
    def body(dp_ref, t_ref, dinv_ref, g_ref):
        deg = jnp.sum(dp_ref[...], axis=0, keepdims=True) + 1.0
        dinv = jax.lax.rsqrt(deg)
        dinv_ref[...] = dinv
        g_ref[...] = t_ref[...] * dinv
    return pl.pallas_call(
        body, out_shape=[jax.ShapeDtypeStruct((1, NPAD), jnp.float32),
                         jax.ShapeDtypeStruct((H, NPAD), jnp.float32)])(degp, t1T)


NCB = 2048
NCBLK = NPAD // NCB


def _nblk():
    return pl.BlockSpec((H, NCB), lambda i: (0, i))


def _tk_mid(agg, tT, dinv, b, wT):
    def body(a_ref, t_ref, dinv_ref, b_ref, w_ref, tn_ref, gn_ref):
        dinv = dinv_ref[...]
        h = jnp.maximum(dinv * a_ref[...] + dinv * dinv * t_ref[...]
                        + b_ref[...], 0.0)
        tn = _bdot(w_ref[...], h)
        tn_ref[...] = tn
        gn_ref[...] = tn * dinv
    return pl.pallas_call(
        body, grid=(NCBLK,),
        in_specs=[_nblk(), _nblk(),
                  pl.BlockSpec((1, NCB), lambda i: (0, i)),
                  pl.BlockSpec((H, 1), lambda i: (0, 0)),
                  pl.BlockSpec((H, H), lambda i: (0, 0))],
        out_specs=[_nblk(), _nblk()],
        out_shape=[jax.ShapeDtypeStruct((H, NPAD), jnp.float32),
                   jax.ShapeDtypeStruct((H, NPAD), jnp.float32)])(
        agg, tT, dinv, b, wT)


def _tk_node_tables(agg, tT, dinv, b, waT, wbT):
    def body(a_ref, t_ref, dinv_ref, b_ref, wa_ref, wb_ref, oa_ref, ob_ref):
        dinv = dinv_ref[...]
        node = dinv * a_ref[...] + dinv * dinv * t_ref[...] + b_ref[...]
        oa_ref[...] = _bdot_t(node, wa_ref[...])
        ob_ref[...] = _bdot_t(node, wb_ref[...])
    return pl.pallas_call(
        body, grid=(NCBLK,),
        in_specs=[_nblk(), _nblk(),
                  pl.BlockSpec((1, NCB), lambda i: (0, i)),
                  pl.BlockSpec((H, 1), lambda i: (0, 0)),
                  pl.BlockSpec((H, H), lambda i: (0, 0)),
                  pl.BlockSpec((H, H), lambda i: (0, 0))],
        out_specs=[pl.BlockSpec((NCB, H), lambda i: (i, 0)),
                   pl.BlockSpec((NCB, H), lambda i: (i, 0))],
        out_shape=[jax.ShapeDtypeStruct((NPAD, H), jnp.float32),
                   jax.ShapeDtypeStruct((NPAD, H), jnp.float32)])(
        agg, tT, dinv, b, waT, wbT)


BE2 = 4000
NB2 = E // BE2


def _row_spec():
    return pl.BlockSpec((1, H), lambda i: (0, 0))


def _eblk():
    return pl.BlockSpec((BE2, H), lambda i: (i, 0))


def _p1_body(sa_ref, sb_ref, ew_ref, c_ref, b_ref, h_ref, s_ref, q_ref):
    i = pl.program_id(0)
    ew16 = ew_ref[...].astype(jnp.bfloat16).astype(jnp.float32)
    c16 = c_ref[...].astype(jnp.bfloat16).astype(jnp.float32)
    h = sa_ref[...] + sb_ref[...] + ew16 * c16 + b_ref[...]
    h_ref[...] = h

    @pl.when(i == 0)
    def _():
        s_ref[...] = jnp.zeros_like(s_ref)
        q_ref[...] = jnp.zeros_like(q_ref)

    s_ref[...] += jnp.sum(h, axis=0, keepdims=True)
    q_ref[...] += jnp.sum(h * h, axis=0, keepdims=True)


def _p23_body(h_ref, s_in, q_in, g_ref, be_ref, w_ref, b_ref,
              out_ref, s_ref, q_ref):
    i = pl.program_id(0)
    mu = s_in[...] / E
    var = q_in[...] / E - mu * mu
    a = g_ref[...] * jax.lax.rsqrt(var + 1e-5)
    sh = be_ref[...] - mu * a
    e = jnp.maximum(h_ref[...] * a + sh, 0.0)
    h2 = _bdot(e, w_ref[...]) + b_ref[...]
    out_ref[...] = h2

    @pl.when(i == 0)
    def _():
        s_ref[...] = jnp.zeros_like(s_ref)
        q_ref[...] = jnp.zeros_like(q_ref)

    s_ref[...] += jnp.sum(h2, axis=0, keepdims=True)
    q_ref[...] += jnp.sum(h2 * h2, axis=0, keepdims=True)


def _p4_body(h_ref, s_in, q_in, g_ref, be_ref, w_ref, b0_ref, out_ref):
    mu = s_in[...] / E
    var = q_in[...] / E - mu * mu
    a = g_ref[...] * jax.lax.rsqrt(var + 1e-5)
    sh = be_ref[...] - mu * a
    e = jnp.maximum(h_ref[...] * a + sh, 0.0)
    e16 = e.astype(jnp.bfloat16).astype(jnp.float32)
    w16 = w_ref[...].astype(jnp.bfloat16).astype(jnp.float32)
    out_ref[...] = (jnp.sum(e16 * w16, axis=1, keepdims=True)
                    + b0_ref[...])


def _edge_mlp(SA, SB, ew_col, M1c, M1b, g1, be1, M2w, M2b, g2, be2,
              M3w, M3b, g3, be3, M4w, M4b):
    f32 = jnp.float32
    h1, s1, q1 = pl.pallas_call(
        _p1_body,
        grid=(NB2,),
        in_specs=[_eblk(), _eblk(),
                  pl.BlockSpec((BE2, 1), lambda i: (i, 0)),
                  _row_spec(), _row_spec()],
        out_specs=[_eblk(), _row_spec(), _row_spec()],
        out_shape=[jax.ShapeDtypeStruct((E, H), f32),
                   jax.ShapeDtypeStruct((1, H), f32),
                   jax.ShapeDtypeStruct((1, H), f32)],
    )(SA, SB, ew_col, M1c, M1b)

    def mid(h, s, q, g, be, w, b):
        return pl.pallas_call(
            _p23_body,
            grid=(NB2,),
            in_specs=[_eblk(), _row_spec(), _row_spec(), _row_spec(),
                      _row_spec(), pl.BlockSpec((H, H), lambda i: (0, 0)),
                      _row_spec()],
            out_specs=[_eblk(), _row_spec(), _row_spec()],
            out_shape=[jax.ShapeDtypeStruct((E, H), f32),
                       jax.ShapeDtypeStruct((1, H), f32),
                       jax.ShapeDtypeStruct((1, H), f32)],
        )(h, s, q, g, be, w, b)

    h2, s2, q2 = mid(h1, s1, q1, g1, be1, M2w, M2b)
    h3, s3, q3 = mid(h2, s2, q2, g2, be2, M3w, M3b)

    logits = pl.pallas_call(
        _p4_body,
        grid=(NB2,),
        in_specs=[_eblk(), _row_spec(), _row_spec(), _row_spec(),
                  _row_spec(), _row_spec(),
                  pl.BlockSpec((1, 1), lambda i: (0, 0))],
        out_specs=pl.BlockSpec((BE2, 1), lambda i: (i, 0)),
        out_shape=jax.ShapeDtypeStruct((E, 1), f32),
    )(h3, s3, q3, g3, be3, M4w, M4b)
    return logits


def kernel(x, edge_index, edge_weight, W1, b1, W2, b2, W3, b3,
           M1w, M1b, g1, be1, M2w, M2b, g2, be2, M3w, M3b, g3, be3,
           M4w, M4b):
    src, dst = edge_index[0], edge_index[1]
    ew = edge_weight

    xT = jnp.pad(x.T, ((0, 0), (0, NPAD - N)))
    W1T, W2T, W3T = W1.T, W2.T, W3.T

    degp = _deg_partials(dst, ew)
    t1T = _tk_matmul(W1T, xT)
    dinv, g1T = _tk_dinv(degp, t1T)

    agg1 = _sc_aggregate(g1T, src, dst, ew)
    t2T, g2T = _tk_mid(agg1, t1T, dinv, b1.reshape(H, 1), W2T)
    agg2 = _sc_aggregate(g2T, src, dst, ew)
    t3T, g3T = _tk_mid(agg2, t2T, dinv, b2.reshape(H, 1), W3T)
    agg3 = _sc_aggregate(g3T, src, dst, ew)
    nodeA, nodeB = _tk_node_tables(agg3, t3T, dinv, b3.reshape(H, 1),
                                   M1w[:H], M1w[H:2 * H])

    SA = _sc_edge_rows(nodeA, src)
    SB = _sc_edge_rows(nodeB, dst)

    logits = _edge_mlp(
        SA, SB, ew.reshape(E, 1), M1w[2 * H].reshape(1, H), M1b.reshape(1, H),
        g1.reshape(1, H), be1.reshape(1, H), M2w, M2b.reshape(1, H),
        g2.reshape(1, H), be2.reshape(1, H), M3w, M3b.reshape(1, H),
        g3.reshape(1, H), be3.reshape(1, H), M4w.reshape(1, H),
        M4b.reshape(1, 1))
    return logits.reshape(E)
